# Initial kernel scaffold; baseline (speedup 1.0000x reference)
import jax
import jax.numpy as jnp
from jax.experimental import pallas as pl


def kernel(x, pos_emb):
    raise NotImplementedError("write your pallas kernel here")


# speedup vs baseline: 1.4934x; 1.4934x over previous
import jax
import jax.numpy as jnp
from jax.experimental import pallas as pl


BLOCK_L = 512


def _add_kernel(x_ref, pos_ref, out_ref):
    out_ref[...] = x_ref[...] + pos_ref[...]


def kernel(x, pos_emb):
    B, L, D = x.shape
    nl = L // BLOCK_L
    return pl.pallas_call(
        _add_kernel,
        grid=(nl, B),
        in_specs=[
            pl.BlockSpec((1, BLOCK_L, D), lambda l, b: (b, l, 0)),
            pl.BlockSpec((BLOCK_L, D), lambda l, b: (l, 0)),
        ],
        out_specs=pl.BlockSpec((1, BLOCK_L, D), lambda l, b: (b, l, 0)),
        out_shape=jax.ShapeDtypeStruct((B, L, D), x.dtype),
    )(x, pos_emb)


# speedup vs baseline: 1.6685x; 1.1173x over previous
import jax
import jax.numpy as jnp
from jax.experimental import pallas as pl


BLOCK_L = 1024


def _add_kernel(x_ref, pos_ref, out_ref):
    out_ref[...] = x_ref[...] + pos_ref[...]


def kernel(x, pos_emb):
    B, L, D = x.shape
    nl = L // BLOCK_L
    return pl.pallas_call(
        _add_kernel,
        grid=(nl, B),
        in_specs=[
            pl.BlockSpec((1, BLOCK_L, D), lambda l, b: (b, l, 0)),
            pl.BlockSpec((BLOCK_L, D), lambda l, b: (l, 0)),
        ],
        out_specs=pl.BlockSpec((1, BLOCK_L, D), lambda l, b: (b, l, 0)),
        out_shape=jax.ShapeDtypeStruct((B, L, D), x.dtype),
    )(x, pos_emb)


# speedup vs baseline: 1.7406x; 1.0432x over previous
import jax
import jax.numpy as jnp
from jax.experimental import pallas as pl


BLOCK_L = 2048


def _add_kernel(x_ref, pos_ref, out_ref):
    out_ref[...] = x_ref[...] + pos_ref[...]


def kernel(x, pos_emb):
    B, L, D = x.shape
    nl = L // BLOCK_L
    return pl.pallas_call(
        _add_kernel,
        grid=(nl, B),
        in_specs=[
            pl.BlockSpec((1, BLOCK_L, D), lambda l, b: (b, l, 0)),
            pl.BlockSpec((BLOCK_L, D), lambda l, b: (l, 0)),
        ],
        out_specs=pl.BlockSpec((1, BLOCK_L, D), lambda l, b: (b, l, 0)),
        out_shape=jax.ShapeDtypeStruct((B, L, D), x.dtype),
    )(x, pos_emb)
